# baseline (speedup 1.0000x reference)
import functools

import jax
import jax.numpy as jnp
from jax import lax
from jax.experimental import pallas as pl
from jax.experimental.pallas import tpu as pltpu
from jax.experimental.pallas import tpu_sc as plsc

B, H, S = 1, 12, 2048
_L = 16
_NC, _NS = 2, 16
_NW = _NC * _NS
_RPW = S // _NW
_NG = _RPW // _L
_CH = _L * S


def _phase1_kernel(xs_ref, w_ref, b_ref, g_ref):
    m = jnp.max(xs_ref[0], axis=1)
    a = jax.lax.dot_general(
        m[None, :], w_ref[...],
        dimension_numbers=(((1,), (1,)), ((), ())),
        preferred_element_type=jnp.float32,
    )
    a = a + b_ref[...]
    g_ref[...] = 0.5 * a * (1.0 + jax.lax.erf(a * 0.7071067811865476))


_NCHUNK = H * _NG
_NSLOT = 3


def _phase2_sc(x_hbm, g_hbm, out_hbm, b0, b1, b2, gv, in_sems, out_sems):
    bufs = (b0, b1, b2)
    wid = lax.axis_index("s") * _NC + lax.axis_index("c")
    base_s = wid * _RPW
    pltpu.sync_copy(g_hbm.at[pl.ds(base_s, _RPW)], gv)
    lane = lax.iota(jnp.int32, _L)
    row_base = lane * S

    def chunk_base(k):
        h, grp = divmod(k, _NG)
        return (h * S + base_s + grp * _L) * S

    def in_copy(k, slot):
        return pltpu.make_async_copy(
            x_hbm.at[pl.ds(chunk_base(k), _CH)], bufs[slot],
            in_sems.at[slot])

    def out_copy(k, slot):
        return pltpu.make_async_copy(
            bufs[slot], out_hbm.at[pl.ds(chunk_base(k), _CH)],
            out_sems.at[slot])

    in_copy(0, 0).start()
    for k in range(_NCHUNK):
        slot = k % _NSLOT
        s1 = (k + 1) % _NSLOT
        if k + 1 < _NCHUNK:
            if k + 1 >= _NSLOT:
                out_copy(k + 1 - _NSLOT, s1).wait()
            in_copy(k + 1, s1).start()
        in_copy(k, slot).wait()
        bslot = bufs[slot]
        cm = plsc.load_gather(bslot, [row_base])
        best = row_base
        idxv = row_base + 1

        def col_body(j, c):
            cm, best, idxv = c
            v = plsc.load_gather(bslot, [idxv])
            p = v > cm
            return (jnp.where(p, v, cm), jnp.where(p, idxv, best), idxv + 1)

        cm, best, _ = lax.fori_loop(1, S, col_body, (cm, best, idxv))
        gvals = gv[pl.ds((k % _NG) * _L, _L)]
        plsc.store_scatter(bslot, [best], gvals)
        out_copy(k, slot).start()
    for k in range(_NCHUNK - _NSLOT, _NCHUNK):
        out_copy(k, k % _NSLOT).wait()


def kernel(x, W, b):
    x3 = x.reshape(H, S, S)
    g = pl.pallas_call(
        _phase1_kernel,
        grid=(1,),
        in_specs=[
            pl.BlockSpec((1, S, S), lambda i: (H - 1, 0, 0)),
            pl.BlockSpec((S, S), lambda i: (0, 0)),
            pl.BlockSpec((1, S), lambda i: (0, 0)),
        ],
        out_specs=pl.BlockSpec((1, S), lambda i: (0, 0)),
        out_shape=jax.ShapeDtypeStruct((1, S), jnp.float32),
    )(x3, W, b.reshape(1, S))

    phase2 = functools.partial(
        pl.kernel,
        mesh=plsc.VectorSubcoreMesh(core_axis_name="c", subcore_axis_name="s"),
        out_type=jax.ShapeDtypeStruct((H * S * S,), jnp.float32),
        scratch_types=[
            pltpu.VMEM((_CH,), jnp.float32),
            pltpu.VMEM((_CH,), jnp.float32),
            pltpu.VMEM((_CH,), jnp.float32),
            pltpu.VMEM((_RPW,), jnp.float32),
            pltpu.SemaphoreType.DMA((_NSLOT,)),
            pltpu.SemaphoreType.DMA((_NSLOT,)),
        ],
        compiler_params=pltpu.CompilerParams(needs_layout_passes=False),
    )(_phase2_sc)
    out = phase2(x.reshape(H * S * S), g.reshape(S))
    return out.reshape(B, H, S, S)


# speedup vs baseline: 2.7725x; 2.7725x over previous
import functools

import jax
import jax.numpy as jnp
from jax import lax
from jax.experimental import pallas as pl
from jax.experimental.pallas import tpu as pltpu
from jax.experimental.pallas import tpu_sc as plsc

B, H, S = 1, 12, 2048
_ROWS = 256
_L = 16
_NC, _NS = 2, 16
_NW = _NC * _NS
_RPW = S // _NW


def _phase1_kernel(xs_ref, w_ref, b_ref, g_ref):
    m = jnp.max(xs_ref[0], axis=1)
    a = jax.lax.dot_general(
        m[None, :], w_ref[...],
        dimension_numbers=(((1,), (1,)), ((), ())),
        preferred_element_type=jnp.float32,
    )
    a = a + b_ref[...]
    g_ref[...] = 0.5 * a * (1.0 + jax.lax.erf(a * 0.7071067811865476))


def _phase2_kernel(x_ref, o_ref, i_ref):
    blk = x_ref[0]
    m = jnp.max(blk, axis=-1, keepdims=True)
    lane = jax.lax.broadcasted_iota(jnp.int32, blk.shape, 1)
    cand = jnp.where(blk == m, lane, S)
    i_ref[...] = jnp.min(cand, axis=-1)
    o_ref[0] = blk


def _phase3_sc(idx_hbm, g_hbm, out_ref, idxv, posv, gvals, sem):
    wid = lax.axis_index("s") * _NC + lax.axis_index("c")
    base_s = wid * _RPW
    pltpu.sync_copy(g_hbm.at[pl.ds(base_s, _RPW)], gvals)
    lane = lax.iota(jnp.int32, _L)
    for h in range(H):
        r0 = h * S + base_s
        pltpu.sync_copy(idx_hbm.at[pl.ds(r0, _RPW)], idxv)
        for q in range(_RPW // _L):
            iv = idxv[pl.ds(q * _L, _L)]
            posv[pl.ds(q * _L, _L)] = (r0 + q * _L + lane) * S + iv
        pltpu.async_copy(gvals, out_ref.at[posv], sem).wait()


def kernel(x, W, b):
    x3 = x.reshape(H, S, S)
    g = pl.pallas_call(
        _phase1_kernel,
        grid=(1,),
        in_specs=[
            pl.BlockSpec((1, S, S), lambda i: (H - 1, 0, 0)),
            pl.BlockSpec((S, S), lambda i: (0, 0)),
            pl.BlockSpec((1, S), lambda i: (0, 0)),
        ],
        out_specs=pl.BlockSpec((1, S), lambda i: (0, 0)),
        out_shape=jax.ShapeDtypeStruct((1, S), jnp.float32),
    )(x3, W, b.reshape(1, S))

    out2, idx = pl.pallas_call(
        _phase2_kernel,
        grid=(H, S // _ROWS),
        in_specs=[
            pl.BlockSpec((1, _ROWS, S), lambda h, i: (h, i, 0)),
        ],
        out_specs=[
            pl.BlockSpec((1, _ROWS, S), lambda h, i: (h, i, 0)),
            pl.BlockSpec((_ROWS,), lambda h, i: (h * (S // _ROWS) + i,)),
        ],
        out_shape=[
            jax.ShapeDtypeStruct((H, S, S), jnp.float32),
            jax.ShapeDtypeStruct((H * S,), jnp.int32),
        ],
        compiler_params=pltpu.CompilerParams(
            dimension_semantics=("parallel", "parallel"),
        ),
    )(x3)

    scatter = functools.partial(
        pl.kernel,
        mesh=plsc.VectorSubcoreMesh(core_axis_name="c", subcore_axis_name="s"),
        out_type=(),
        scratch_types=[
            pltpu.VMEM((_RPW,), jnp.int32),
            pltpu.VMEM((_RPW,), jnp.int32),
            pltpu.VMEM((_RPW,), jnp.float32),
            pltpu.SemaphoreType.DMA,
        ],
        compiler_params=pltpu.CompilerParams(needs_layout_passes=False),
    )(_phase3_sc)

    o_ref = jax.new_ref(out2.reshape(H * S * S))
    scatter(idx, g.reshape(S), o_ref)
    return o_ref[...].reshape(B, H, S, S)


# speedup vs baseline: 10.3293x; 3.7257x over previous
import jax
import jax.numpy as jnp
from jax.experimental import pallas as pl
from jax.experimental.pallas import tpu as pltpu

B, H, S = 1, 12, 2048
_ROWS = 512


def _phase1_kernel(xs_ref, w_ref, b_ref, g_ref):
    m = jnp.max(xs_ref[0], axis=1)
    a = jax.lax.dot_general(
        m[None, :], w_ref[...],
        dimension_numbers=(((1,), (1,)), ((), ())),
        preferred_element_type=jnp.float32,
    )
    a = a + b_ref[...]
    g_ref[...] = 0.5 * a * (1.0 + jax.lax.erf(a * 0.7071067811865476))


def _phase2_kernel(x_ref, g_ref, o_ref):
    blk = x_ref[0]
    m = jnp.max(blk, axis=-1, keepdims=True)
    lane = jax.lax.broadcasted_iota(jnp.int32, blk.shape, 1)
    cand = jnp.where(blk == m, lane, S)
    idx = jnp.min(cand, axis=-1, keepdims=True)
    o_ref[0] = jnp.where(lane == idx, g_ref[0][:, None], blk)


def kernel(x, W, b):
    x3 = x.reshape(H, S, S)
    g = pl.pallas_call(
        _phase1_kernel,
        grid=(1,),
        in_specs=[
            pl.BlockSpec((1, S, S), lambda i: (H - 1, 0, 0)),
            pl.BlockSpec((S, S), lambda i: (0, 0)),
            pl.BlockSpec((1, S), lambda i: (0, 0)),
        ],
        out_specs=pl.BlockSpec((1, S), lambda i: (0, 0)),
        out_shape=jax.ShapeDtypeStruct((1, S), jnp.float32),
    )(x3, W, b.reshape(1, S))

    out = pl.pallas_call(
        _phase2_kernel,
        grid=(H, S // _ROWS),
        in_specs=[
            pl.BlockSpec((1, _ROWS, S), lambda h, i: (h, i, 0)),
            pl.BlockSpec((1, _ROWS), lambda h, i: (0, i)),
        ],
        out_specs=pl.BlockSpec((1, _ROWS, S), lambda h, i: (h, i, 0)),
        out_shape=jax.ShapeDtypeStruct((H, S, S), jnp.float32),
        compiler_params=pltpu.CompilerParams(
            dimension_semantics=("parallel", "parallel"),
        ),
    )(x3, g)
    return out.reshape(B, H, S, S)


# speedup vs baseline: 10.6238x; 1.0285x over previous
import jax
import jax.numpy as jnp
from jax.experimental import pallas as pl
from jax.experimental.pallas import tpu as pltpu

B, H, S = 1, 12, 2048
_ROWS = 1024


def _phase1_kernel(xs_ref, w_ref, b_ref, g_ref):
    m = jnp.max(xs_ref[0], axis=1)
    a = jax.lax.dot_general(
        m[None, :], w_ref[...],
        dimension_numbers=(((1,), (1,)), ((), ())),
        preferred_element_type=jnp.float32,
    )
    a = a + b_ref[...]
    g_ref[...] = 0.5 * a * (1.0 + jax.lax.erf(a * 0.7071067811865476))


def _phase2_kernel(x_ref, g_ref, o_ref):
    blk = x_ref[0]
    m = jnp.max(blk, axis=-1, keepdims=True)
    lane = jax.lax.broadcasted_iota(jnp.int32, blk.shape, 1)
    cand = jnp.where(blk == m, lane, S)
    idx = jnp.min(cand, axis=-1, keepdims=True)
    o_ref[0] = jnp.where(lane == idx, g_ref[0][:, None], blk)


def kernel(x, W, b):
    x3 = x.reshape(H, S, S)
    g = pl.pallas_call(
        _phase1_kernel,
        grid=(1,),
        in_specs=[
            pl.BlockSpec((1, S, S), lambda i: (H - 1, 0, 0)),
            pl.BlockSpec((S, S), lambda i: (0, 0)),
            pl.BlockSpec((1, S), lambda i: (0, 0)),
        ],
        out_specs=pl.BlockSpec((1, S), lambda i: (0, 0)),
        out_shape=jax.ShapeDtypeStruct((1, S), jnp.float32),
    )(x3, W, b.reshape(1, S))

    out = pl.pallas_call(
        _phase2_kernel,
        grid=(H, S // _ROWS),
        in_specs=[
            pl.BlockSpec((1, _ROWS, S), lambda h, i: (h, i, 0)),
            pl.BlockSpec((1, _ROWS), lambda h, i: (0, i)),
        ],
        out_specs=pl.BlockSpec((1, _ROWS, S), lambda h, i: (h, i, 0)),
        out_shape=jax.ShapeDtypeStruct((H, S, S), jnp.float32),
        compiler_params=pltpu.CompilerParams(
            dimension_semantics=("parallel", "parallel"),
        ),
    )(x3, g)
    return out.reshape(B, H, S, S)


# speedup vs baseline: 10.7808x; 1.0148x over previous
import jax
import jax.numpy as jnp
from jax.experimental import pallas as pl
from jax.experimental.pallas import tpu as pltpu

B, H, S = 1, 12, 2048
_R = 1024
_NB = H * S // _R


def _fused_kernel(x_ref, w_ref, b_ref, o_ref, m_sc, g_sc):
    i = pl.program_id(0)
    blk = x_ref[0]

    @pl.when(i < 2)
    def _prologue():
        m_sc[0, pl.ds(i * _R, _R)] = jnp.max(blk, axis=-1)

    @pl.when(i == 1)
    def _linear():
        a = jax.lax.dot_general(
            m_sc[...], w_ref[...],
            dimension_numbers=(((1,), (1,)), ((), ())),
            preferred_element_type=jnp.float32,
        ) + b_ref[...]
        g_sc[...] = 0.5 * a * (1.0 + jax.lax.erf(a * 0.7071067811865476))

    @pl.when(i >= 2)
    def _stream():
        m = jnp.max(blk, axis=-1, keepdims=True)
        lane = jax.lax.broadcasted_iota(jnp.int32, blk.shape, 1)
        cand = jnp.where(blk == m, lane, S)
        idx = jnp.min(cand, axis=-1, keepdims=True)
        gv = g_sc[0, pl.ds(((i - 2) % 2) * _R, _R)]
        o_ref[0] = jnp.where(lane == idx, gv[:, None], blk)


def kernel(x, W, b):
    x4 = x.reshape(_NB, _R, S)
    out = pl.pallas_call(
        _fused_kernel,
        grid=(_NB + 2,),
        in_specs=[
            pl.BlockSpec((1, _R, S),
                         lambda i: (jnp.where(i < 2, i + _NB - 2, i - 2), 0, 0)),
            pl.BlockSpec((S, S), lambda i: (0, 0)),
            pl.BlockSpec((1, S), lambda i: (0, 0)),
        ],
        out_specs=pl.BlockSpec((1, _R, S),
                               lambda i: (jnp.maximum(i - 2, 0), 0, 0)),
        out_shape=jax.ShapeDtypeStruct((_NB, _R, S), jnp.float32),
        scratch_shapes=[
            pltpu.VMEM((1, S), jnp.float32),
            pltpu.VMEM((1, S), jnp.float32),
        ],
    )(x4, W, b.reshape(1, S))
    return out.reshape(B, H, S, S)
